# baseline (speedup 1.0000x reference)
import functools

import jax
import jax.numpy as jnp
from jax import lax
from jax.experimental import pallas as pl
from jax.experimental.pallas import tpu as pltpu
from jax.experimental.pallas import tpu_sc as plsc

N = 10000
C = 256
CW = C // 2
K = 16
NC = 2
NS = 16
NW = NC * NS
NPAD = 10240
NPW = NPAD // NW
CHUNK = 16
NBUF = 2
NCHUNKS = NPW // CHUNK
L = 16


def _sc_gather_sum(x_packed, idx_flat):
    mesh = plsc.VectorSubcoreMesh(core_axis_name="c", subcore_axis_name="s")

    @functools.partial(
        pl.kernel,
        mesh=mesh,
        compiler_params=pltpu.CompilerParams(needs_layout_passes=False),
        out_type=jax.ShapeDtypeStruct((NPAD, CW), jnp.int32),
        scratch_types=[
            pltpu.VMEM((NPW * K,), jnp.int32),
            pltpu.VMEM((NBUF, CHUNK * K, CW), jnp.int32),
            pltpu.VMEM((NBUF, CHUNK, CW), jnp.int32),
            pltpu.SemaphoreType.DMA,
            pltpu.SemaphoreType.DMA,
            pltpu.SemaphoreType.DMA,
            pltpu.SemaphoreType.DMA,
        ],
    )
    def k(xt_hbm, idx_hbm, out_hbm, idx_v, gbuf, obuf, gs0, gs1, os0, os1):
        gsems = [gs0, gs1]
        osems = [os0, os1]
        wid = lax.axis_index("s") * NC + lax.axis_index("c")
        base = wid * NPW
        pltpu.sync_copy(idx_hbm.at[pl.ds(base * K, NPW * K)], idx_v)

        def gather(chunk_i, buf_i):
            pltpu.async_copy(
                xt_hbm.at[idx_v.at[pl.ds(chunk_i * (CHUNK * K), CHUNK * K)]],
                gbuf.at[buf_i],
                gsems[buf_i],
            )

        def flush(chunk_i, buf_i):
            pltpu.async_copy(
                obuf.at[buf_i],
                out_hbm.at[pl.ds(base + chunk_i * CHUNK, CHUNK)],
                osems[buf_i],
            )

        def flush_wait(buf_i):
            pltpu.make_async_copy(
                obuf.at[buf_i],
                out_hbm.at[pl.ds(base, CHUNK)],
                osems[buf_i],
            ).wait()

        for b in range(NBUF):
            gather(b, b)

        def body(i, _):
            for b in range(NBUF):
                ci = i * NBUF + b
                pltpu.make_async_copy(
                    xt_hbm.at[idx_v.at[pl.ds(0, CHUNK * K)]],
                    gbuf.at[b],
                    gsems[b],
                ).wait()

                @pl.when(ci >= NBUF)
                def _():
                    flush_wait(b)

                def node_body(node, _):
                    r0 = node * K
                    for j in range(CW // L):
                        v = [
                            plsc.bitcast(
                                gbuf[b, r0 + r, pl.ds(j * L, L)],
                                jnp.bfloat16,
                            )
                            for r in range(K)
                        ]
                        while len(v) > 1:
                            v = [v[i] + v[i + 1]
                                 for i in range(0, len(v), 2)]
                        obuf[b, node, pl.ds(j * L, L)] = plsc.bitcast(
                            v[0], jnp.int32)
                    return 0

                lax.fori_loop(0, CHUNK, node_body, 0)

                flush(ci, b)
                nxt = ci + NBUF

                @pl.when(nxt < NCHUNKS)
                def _():
                    gather(nxt, b)
            return 0

        lax.fori_loop(0, NCHUNKS // NBUF, body, 0)
        for b in range(NBUF):
            flush_wait(b)

    return k(x_packed, idx_flat)


def _tc_conv(x_t, agg, W, b2):
    TILE = 512

    def body(xt_ref, agg_ref, w_ref, b_ref, out_ref):
        h = xt_ref[...] + agg_ref[...].astype(jnp.float32)
        acc = lax.dot_general(
            w_ref[...], h, (((1,), (1,)), ((), ())),
            preferred_element_type=jnp.float32,
        )
        out_ref[...] = jnp.maximum(acc + b_ref[...], 0.0)

    return pl.pallas_call(
        body,
        grid=(NPAD // TILE,),
        in_specs=[
            pl.BlockSpec((TILE, C), lambda i: (i, 0)),
            pl.BlockSpec((TILE, C), lambda i: (i, 0)),
            pl.BlockSpec((C, C), lambda i: (0, 0)),
            pl.BlockSpec((C, 1), lambda i: (0, 0)),
        ],
        out_specs=pl.BlockSpec((C, TILE), lambda i: (0, i)),
        out_shape=jax.ShapeDtypeStruct((C, NPAD), jnp.float32),
    )(x_t, agg, W, b2)


def kernel(x, x_0, edge_index, W, b):
    del x_0
    x_flat = x[0, :, :, 0]
    x_t = jnp.transpose(x_flat)
    x_t_pad = jnp.pad(x_t, ((0, NPAD - N), (0, 0)))
    x_bf = x_t_pad.astype(jnp.bfloat16)
    y = x_bf.reshape(NPAD, CW // L, 2, L).transpose(0, 1, 3, 2)
    x_packed = lax.bitcast_convert_type(y, jnp.int32).reshape(NPAD, CW)
    idx = edge_index[0, 0]
    idx_pad = jnp.pad(idx, ((0, NPAD - N), (0, 0))).reshape(NPAD * K)
    agg_w = _sc_gather_sum(x_packed, idx_pad)
    agg = lax.bitcast_convert_type(
        agg_w.reshape(NPAD, CW // L, L), jnp.bfloat16
    ).transpose(0, 1, 3, 2).reshape(NPAD, C)
    out = _tc_conv(x_t_pad, agg, W, jnp.reshape(b, (C, 1)))
    return out[:, :N][None, :, :, None]


# speedup vs baseline: 1.0050x; 1.0050x over previous
import functools

import jax
import jax.numpy as jnp
from jax import lax
from jax.experimental import pallas as pl
from jax.experimental.pallas import tpu as pltpu
from jax.experimental.pallas import tpu_sc as plsc

N = 10000
C = 256
CW = C // 2
K = 16
NC = 2
NS = 16
NW = NC * NS
NPAD = 10240
NPW = NPAD // NW
CHUNK = 8
NBUF = 4
NCHUNKS = NPW // CHUNK
L = 16


def _sc_gather_sum(x_packed, idx_flat):
    mesh = plsc.VectorSubcoreMesh(core_axis_name="c", subcore_axis_name="s")

    @functools.partial(
        pl.kernel,
        mesh=mesh,
        compiler_params=pltpu.CompilerParams(needs_layout_passes=False),
        out_type=jax.ShapeDtypeStruct((NPAD, CW), jnp.int32),
        scratch_types=[
            pltpu.VMEM((NPW * K,), jnp.int32),
            pltpu.VMEM((NBUF, CHUNK * K, CW), jnp.int32),
            pltpu.VMEM((NBUF, CHUNK, CW), jnp.int32),
            pltpu.SemaphoreType.DMA,
            pltpu.SemaphoreType.DMA,
            pltpu.SemaphoreType.DMA,
            pltpu.SemaphoreType.DMA,
            pltpu.SemaphoreType.DMA,
            pltpu.SemaphoreType.DMA,
            pltpu.SemaphoreType.DMA,
            pltpu.SemaphoreType.DMA,
        ],
    )
    def k(xt_hbm, idx_hbm, out_hbm, idx_v, gbuf, obuf,
          gs0, gs1, gs2, gs3, os0, os1, os2, os3):
        gsems = [gs0, gs1, gs2, gs3]
        osems = [os0, os1, os2, os3]
        wid = lax.axis_index("s") * NC + lax.axis_index("c")
        base = wid * NPW
        pltpu.sync_copy(idx_hbm.at[pl.ds(base * K, NPW * K)], idx_v)

        def gather(chunk_i, buf_i):
            pltpu.async_copy(
                xt_hbm.at[idx_v.at[pl.ds(chunk_i * (CHUNK * K), CHUNK * K)]],
                gbuf.at[buf_i],
                gsems[buf_i],
            )

        def flush(chunk_i, buf_i):
            pltpu.async_copy(
                obuf.at[buf_i],
                out_hbm.at[pl.ds(base + chunk_i * CHUNK, CHUNK)],
                osems[buf_i],
            )

        def flush_wait(buf_i):
            pltpu.make_async_copy(
                obuf.at[buf_i],
                out_hbm.at[pl.ds(base, CHUNK)],
                osems[buf_i],
            ).wait()

        for b in range(NBUF):
            gather(b, b)

        def body(i, _):
            for b in range(NBUF):
                ci = i * NBUF + b
                pltpu.make_async_copy(
                    xt_hbm.at[idx_v.at[pl.ds(0, CHUNK * K)]],
                    gbuf.at[b],
                    gsems[b],
                ).wait()

                @pl.when(ci >= NBUF)
                def _():
                    flush_wait(b)

                def node_body(node, _):
                    r0 = node * K
                    for j in range(CW // L):
                        v = [
                            plsc.bitcast(
                                gbuf[b, r0 + r, pl.ds(j * L, L)],
                                jnp.bfloat16,
                            )
                            for r in range(K)
                        ]
                        while len(v) > 1:
                            v = [v[i] + v[i + 1]
                                 for i in range(0, len(v), 2)]
                        obuf[b, node, pl.ds(j * L, L)] = plsc.bitcast(
                            v[0], jnp.int32)
                    return 0

                lax.fori_loop(0, CHUNK, node_body, 0)

                flush(ci, b)
                nxt = ci + NBUF

                @pl.when(nxt < NCHUNKS)
                def _():
                    gather(nxt, b)
            return 0

        lax.fori_loop(0, NCHUNKS // NBUF, body, 0)
        for b in range(NBUF):
            flush_wait(b)

    return k(x_packed, idx_flat)


def _tc_conv(x_t, agg, W, b2):
    TILE = 512

    def body(xt_ref, agg_ref, w_ref, b_ref, out_ref):
        h = xt_ref[...] + agg_ref[...].astype(jnp.float32)
        acc = lax.dot_general(
            w_ref[...], h, (((1,), (1,)), ((), ())),
            preferred_element_type=jnp.float32,
        )
        out_ref[...] = jnp.maximum(acc + b_ref[...], 0.0)

    return pl.pallas_call(
        body,
        grid=(NPAD // TILE,),
        in_specs=[
            pl.BlockSpec((TILE, C), lambda i: (i, 0)),
            pl.BlockSpec((TILE, C), lambda i: (i, 0)),
            pl.BlockSpec((C, C), lambda i: (0, 0)),
            pl.BlockSpec((C, 1), lambda i: (0, 0)),
        ],
        out_specs=pl.BlockSpec((C, TILE), lambda i: (0, i)),
        out_shape=jax.ShapeDtypeStruct((C, NPAD), jnp.float32),
    )(x_t, agg, W, b2)


def kernel(x, x_0, edge_index, W, b):
    del x_0
    x_flat = x[0, :, :, 0]
    x_t = jnp.transpose(x_flat)
    x_t_pad = jnp.pad(x_t, ((0, NPAD - N), (0, 0)))
    x_bf = x_t_pad.astype(jnp.bfloat16)
    y = x_bf.reshape(NPAD, CW // L, 2, L).transpose(0, 1, 3, 2)
    x_packed = lax.bitcast_convert_type(y, jnp.int32).reshape(NPAD, CW)
    idx = edge_index[0, 0]
    idx_pad = jnp.pad(idx, ((0, NPAD - N), (0, 0))).reshape(NPAD * K)
    agg_w = _sc_gather_sum(x_packed, idx_pad)
    agg = lax.bitcast_convert_type(
        agg_w.reshape(NPAD, CW // L, L), jnp.bfloat16
    ).transpose(0, 1, 3, 2).reshape(NPAD, C)
    out = _tc_conv(x_t_pad, agg, W, jnp.reshape(b, (C, 1)))
    return out[:, :N][None, :, :, None]


# speedup vs baseline: 1.0284x; 1.0233x over previous
import functools

import jax
import jax.numpy as jnp
from jax import lax
from jax.experimental import pallas as pl
from jax.experimental.pallas import tpu as pltpu
from jax.experimental.pallas import tpu_sc as plsc

N = 10000
C = 256
CW = C // 2
K = 16
NC = 2
NS = 16
NPAD = 10240
NPW0 = 496
NPW1 = 144
CHUNK = 8
NBUF = 2
L = 16


def _sc_gather_sum(x_packed, idx_flat):
    mesh = plsc.VectorSubcoreMesh(core_axis_name="c", subcore_axis_name="s")

    @functools.partial(
        pl.kernel,
        mesh=mesh,
        compiler_params=pltpu.CompilerParams(needs_layout_passes=False),
        out_type=jax.ShapeDtypeStruct((NPAD, CW), jnp.int32),
        scratch_types=[
            pltpu.VMEM((NPW0 * K,), jnp.int32),
            pltpu.VMEM((NBUF, CHUNK * K, CW), jnp.int32),
            pltpu.VMEM((NBUF, CHUNK, CW), jnp.int32),
            pltpu.SemaphoreType.DMA,
            pltpu.SemaphoreType.DMA,
            pltpu.SemaphoreType.DMA,
            pltpu.SemaphoreType.DMA,
        ],
    )
    def k(xt_hbm, idx_hbm, out_hbm, idx_v, gbuf, obuf, gs0, gs1, os0, os1):
        gsems = [gs0, gs1]
        osems = [os0, os1]
        s = lax.axis_index("s")

        def worker_body(base, npw):
            nchunks = npw // CHUNK
            pltpu.sync_copy(
                idx_hbm.at[pl.ds(base * K, npw * K)],
                idx_v.at[pl.ds(0, npw * K)],
            )

            def gather(chunk_i, buf_i):
                pltpu.async_copy(
                    xt_hbm.at[
                        idx_v.at[pl.ds(chunk_i * (CHUNK * K), CHUNK * K)]],
                    gbuf.at[buf_i],
                    gsems[buf_i],
                )

            def flush(chunk_i, buf_i):
                pltpu.async_copy(
                    obuf.at[buf_i],
                    out_hbm.at[pl.ds(base + chunk_i * CHUNK, CHUNK)],
                    osems[buf_i],
                )

            def flush_wait(buf_i):
                pltpu.make_async_copy(
                    obuf.at[buf_i],
                    out_hbm.at[pl.ds(base, CHUNK)],
                    osems[buf_i],
                ).wait()

            for b in range(NBUF):
                gather(b, b)

            def body(i, _):
                for b in range(NBUF):
                    ci = i * NBUF + b
                    pltpu.make_async_copy(
                        xt_hbm.at[idx_v.at[pl.ds(0, CHUNK * K)]],
                        gbuf.at[b],
                        gsems[b],
                    ).wait()

                    @pl.when(ci >= NBUF)
                    def _():
                        flush_wait(b)

                    def node_body(node, _):
                        r0 = node * K
                        for j in range(CW // L):
                            v = [
                                plsc.bitcast(
                                    gbuf[b, r0 + r, pl.ds(j * L, L)],
                                    jnp.bfloat16,
                                )
                                for r in range(K)
                            ]
                            while len(v) > 1:
                                v = [v[i] + v[i + 1]
                                     for i in range(0, len(v), 2)]
                            obuf[b, node, pl.ds(j * L, L)] = plsc.bitcast(
                                v[0], jnp.int32)
                        return 0

                    lax.fori_loop(0, CHUNK, node_body, 0)

                    flush(ci, b)
                    nxt = ci + NBUF

                    @pl.when(nxt < nchunks)
                    def _():
                        gather(nxt, b)
                return 0

            lax.fori_loop(0, nchunks // NBUF, body, 0)
            for b in range(NBUF):
                flush_wait(b)

        @pl.when(lax.axis_index("c") == 0)
        def _():
            worker_body(s * NPW0, NPW0)

        @pl.when(lax.axis_index("c") == 1)
        def _():
            worker_body(NS * NPW0 + s * NPW1, NPW1)

    return k(x_packed, idx_flat)


def _tc_conv(x_t, agg, W, b2):
    TILE = 512

    def body(xt_ref, agg_ref, w_ref, b_ref, out_ref):
        h = xt_ref[...] + agg_ref[...].astype(jnp.float32)
        acc = lax.dot_general(
            w_ref[...], h, (((1,), (1,)), ((), ())),
            preferred_element_type=jnp.float32,
        )
        out_ref[...] = jnp.maximum(acc + b_ref[...], 0.0)

    return pl.pallas_call(
        body,
        grid=(NPAD // TILE,),
        in_specs=[
            pl.BlockSpec((TILE, C), lambda i: (i, 0)),
            pl.BlockSpec((TILE, C), lambda i: (i, 0)),
            pl.BlockSpec((C, C), lambda i: (0, 0)),
            pl.BlockSpec((C, 1), lambda i: (0, 0)),
        ],
        out_specs=pl.BlockSpec((C, TILE), lambda i: (0, i)),
        out_shape=jax.ShapeDtypeStruct((C, NPAD), jnp.float32),
    )(x_t, agg, W, b2)


def kernel(x, x_0, edge_index, W, b):
    del x_0
    x_flat = x[0, :, :, 0]
    x_t = jnp.transpose(x_flat)
    x_t_pad = jnp.pad(x_t, ((0, NPAD - N), (0, 0)))
    x_bf = x_t_pad.astype(jnp.bfloat16)
    y = x_bf.reshape(NPAD, CW // L, 2, L).transpose(0, 1, 3, 2)
    x_packed = lax.bitcast_convert_type(y, jnp.int32).reshape(NPAD, CW)
    idx = edge_index[0, 0]
    idx_pad = jnp.pad(idx, ((0, NPAD - N), (0, 0))).reshape(NPAD * K)
    agg_w = _sc_gather_sum(x_packed, idx_pad)
    agg = lax.bitcast_convert_type(
        agg_w.reshape(NPAD, CW // L, L), jnp.bfloat16
    ).transpose(0, 1, 3, 2).reshape(NPAD, C)
    out = _tc_conv(x_t_pad, agg, W, jnp.reshape(b, (C, 1)))
    return out[:, :N][None, :, :, None]
